# baseline (speedup 1.0000x reference)
import functools

import jax
import jax.numpy as jnp
from jax import lax
from jax.experimental import pallas as pl
from jax.experimental.pallas import tpu as pltpu
from jax.experimental.pallas import tpu_sc as plsc

NUM_TOKENS = 1000000
E_DIM = 64
BATCH = 4096
HIST = 200

NC = 2
NS = 16
NW = NC * NS
N = BATCH * HIST
PER_W = N // NW
CHUNK = 512
NCHUNK = PER_W // CHUNK

_MESH = plsc.VectorSubcoreMesh(core_axis_name="c", subcore_axis_name="s")


@functools.partial(
    pl.kernel,
    mesh=_MESH,
    out_type=jax.ShapeDtypeStruct((N, E_DIM), jnp.float32),
    scratch_types=[
        pltpu.VMEM((2, CHUNK), jnp.int32),
        pltpu.VMEM((2, CHUNK, E_DIM), jnp.float32),
        pltpu.SemaphoreType.DMA((2,)),
    ],
    compiler_params=pltpu.CompilerParams(use_tc_tiling_on_sc=False),
)
def _emb_gather(x_hbm, table_hbm, out_hbm, idx_v, rows_v, gsem):
    wid = lax.axis_index("s") * NC + lax.axis_index("c")
    base = wid * PER_W

    def load_and_fire(chunk, b):
        off = base + chunk * CHUNK
        pltpu.sync_copy(x_hbm.at[pl.ds(off, CHUNK)], idx_v.at[b])
        pltpu.async_copy(table_hbm.at[idx_v.at[b]], rows_v.at[b], gsem.at[b])

    load_and_fire(0, 0)

    def body(i, carry):
        c = i * 2
        for b in range(2):
            cur = c + b
            nxt = cur + 1

            @pl.when(nxt < NCHUNK)
            def _():
                load_and_fire(nxt, 1 - b)

            pltpu.make_async_copy(
                table_hbm.at[idx_v.at[b]], rows_v.at[b], gsem.at[b]
            ).wait()
            pltpu.sync_copy(
                rows_v.at[b], out_hbm.at[pl.ds(base + cur * CHUNK, CHUNK)]
            )
        return carry

    lax.fori_loop(0, NCHUNK // 2, body, 0)


def kernel(x, table):
    xf = x.reshape(-1).astype(jnp.int32)
    out = _emb_gather(xf, table)
    return out.reshape(x.shape[0], x.shape[1], E_DIM)


# speedup vs baseline: 1.0080x; 1.0080x over previous
import functools

import jax
import jax.numpy as jnp
from jax import lax
from jax.experimental import pallas as pl
from jax.experimental.pallas import tpu as pltpu
from jax.experimental.pallas import tpu_sc as plsc

NUM_TOKENS = 1000000
E_DIM = 64
BATCH = 4096
HIST = 200

NC = 2
NS = 16
NW = NC * NS
N = BATCH * HIST
PER_W = N // NW
CHUNK = 256
NCHUNK = PER_W // CHUNK
NBUF = 4
NGROUP = NCHUNK // NBUF

_MESH = plsc.VectorSubcoreMesh(core_axis_name="c", subcore_axis_name="s")


@functools.partial(
    pl.kernel,
    mesh=_MESH,
    out_type=jax.ShapeDtypeStruct((N, E_DIM), jnp.float32),
    scratch_types=[
        pltpu.VMEM((PER_W,), jnp.int32),
        pltpu.VMEM((NBUF, CHUNK, E_DIM), jnp.float32),
        pltpu.SemaphoreType.DMA((NBUF,)),
        pltpu.SemaphoreType.DMA((NBUF,)),
    ],
    compiler_params=pltpu.CompilerParams(use_tc_tiling_on_sc=False),
)
def _emb_gather(x_hbm, table_hbm, out_hbm, idx_v, rows_v, gsem, wsem):
    wid = lax.axis_index("s") * NC + lax.axis_index("c")
    base = wid * PER_W

    pltpu.sync_copy(x_hbm.at[pl.ds(base, PER_W)], idx_v)

    def idx_of(chunk):
        return idx_v.at[pl.ds(chunk * CHUNK, CHUNK)]

    def out_of(chunk):
        return out_hbm.at[pl.ds(base + chunk * CHUNK, CHUNK)]

    def body(g, carry):
        for b in range(NBUF):
            cur = g * NBUF + b

            @pl.when(g > 0)
            def _():
                pltpu.make_async_copy(
                    rows_v.at[b], out_of(cur - NBUF), wsem.at[b]
                ).wait()

            pltpu.async_copy(table_hbm.at[idx_of(cur)], rows_v.at[b], gsem.at[b])
        for b in range(NBUF):
            cur = g * NBUF + b
            pltpu.make_async_copy(
                table_hbm.at[idx_of(cur)], rows_v.at[b], gsem.at[b]
            ).wait()
            pltpu.async_copy(rows_v.at[b], out_of(cur), wsem.at[b])
        return carry

    lax.fori_loop(0, NGROUP, body, 0)

    for b in range(NBUF):
        cur = (NGROUP - 1) * NBUF + b
        pltpu.make_async_copy(rows_v.at[b], out_of(cur), wsem.at[b]).wait()


def kernel(x, table):
    xf = x.reshape(-1).astype(jnp.int32)
    out = _emb_gather(xf, table)
    return out.reshape(x.shape[0], x.shape[1], E_DIM)
